# baseline (speedup 1.0000x reference)
import functools

import jax
import jax.numpy as jnp
from jax import lax
from jax.experimental import pallas as pl
from jax.experimental.pallas import tpu as pltpu
from jax.experimental.pallas import tpu_sc as plsc

N = 10000
E = 320000
D = 128
ED = 16
FP = 881
G = 512

NC = 2
NS = 16
NW = NC * NS
EPW = E // NW
K = 125
NCHUNK = EPW // K
NPS = 640
NPS_LAST = N - (NS - 1) * NPS

_SC_MESH = plsc.VectorSubcoreMesh(core_axis_name="c", subcore_axis_name="s")


NB = 8
LAG = 4


def _seg_partials(table, src3, dst3, nchunk, k, edge_rows=None):
    gather = table is not None
    W = table.shape[-1] if gather else edge_rows.shape[-1]
    assert nchunk * k == EPW and k <= 128
    zeros_nw = jnp.zeros((NPS, W), jnp.float32)
    if edge_rows is None:
        edge_rows = jnp.zeros((8, 16), jnp.float32)
    if table is None:
        table = jnp.zeros((8, 16), jnp.float32)
    @functools.partial(
        pl.kernel,
        mesh=_SC_MESH,
        out_type=jax.ShapeDtypeStruct((NC, N, W), jnp.float32),
        scratch_types=[
            pltpu.VMEM((nchunk, k), jnp.int32),
            pltpu.VMEM((nchunk, k), jnp.int32),
            *[pltpu.VMEM((k, W), jnp.float32) for _ in range(NB)],
            pltpu.VMEM_SHARED((N, W), jnp.float32),
            *[pltpu.SemaphoreType.DMA for _ in range(2 * NB)],
        ],
        compiler_params=pltpu.CompilerParams(use_tc_tiling_on_sc=False),
    )
    def sc_kernel(table_hbm, src_hbm, dst_hbm, zero_hbm, ea_hbm, out_hbm,
                  sidx, didx, *rest):
        rows = rest[:NB]
        acc = rest[NB]
        gsem = rest[NB + 1:NB + 1 + NB]
        ssem = rest[NB + 1 + NB:]
        c = lax.axis_index("c")
        s = lax.axis_index("s")
        wid = s * NC + c

        def for_slices(fn):
            @pl.when(s < NS - 1)
            def _():
                fn(pl.ds(s * NPS, NPS), NPS)

            @pl.when(s == NS - 1)
            def _():
                fn(pl.ds((NS - 1) * NPS, NPS_LAST), NPS_LAST)

        for_slices(lambda sl, n: pltpu.sync_copy(
            zero_hbm.at[pl.ds(0, n)], acc.at[sl]))
        if gather:
            pltpu.sync_copy(src_hbm.at[wid], sidx)
        pltpu.sync_copy(dst_hbm.at[wid], didx)
        plsc.subcore_barrier()

        def fetch_pair(i, b):
            if gather:
                return table_hbm.at[sidx.at[i]], rows[b]
            return ea_hbm.at[pl.ds(wid * EPW + i * k, k)], rows[b]

        def fetch(i, b):
            s_, d_ = fetch_pair(i, b)
            pltpu.async_copy(s_, d_, gsem[b])

        def fetch_wait(i, b):
            s_, d_ = fetch_pair(i, b)
            pltpu.make_async_copy(s_, d_, gsem[b]).wait()

        def scat(i, b):
            pltpu.async_copy(rows[b], acc.at[didx.at[i]], ssem[b], add=True)

        def scat_wait(i, b):
            pltpu.make_async_copy(rows[b], acc.at[didx.at[i]], ssem[b]).wait()

        def process(cc):
            fetch_wait(cc, cc % NB)
            scat(cc, cc % NB)
            if cc + LAG < nchunk:
                scat_wait(cc - LAG, (cc - LAG) % NB)
                fetch(cc + LAG, (cc + LAG) % NB)

        main_iters = (nchunk - 2 * LAG) // NB
        main_end = LAG + main_iters * NB

        for cc in range(NB):
            fetch(cc, cc)
        for cc in range(LAG):
            fetch_wait(cc, cc)
            scat(cc, cc)

        @pl.loop(LAG, main_end, step=NB)
        def _(i0):
            for j in range(NB):
                cc = i0 + j
                b = (LAG + j) % NB
                fetch_wait(cc, b)
                scat(cc, b)
                scat_wait(cc - LAG, j)
                fetch(cc + LAG, j)

        for cc in range(main_end, nchunk):
            process(cc)
        for cc in range(nchunk - 2 * LAG, nchunk):
            scat_wait(cc, cc % NB)

        plsc.subcore_barrier()
        for_slices(lambda sl, n: pltpu.sync_copy(acc.at[sl],
                                                 out_hbm.at[c].at[sl]))

    return sc_kernel(table, src3, dst3, zeros_nw, edge_rows)


def _mm0_body(x_ref, w_ref, o_ref):
    o_ref[...] = jnp.dot(x_ref[...], w_ref[...],
                         preferred_element_type=jnp.float32)


def _combine_body(h_ref, p_ref, ea_ref, wroot_ref, wedge_ref, b_ref,
                  wmsg_ref, h_out, hm_out):
    agg = p_ref[0] + p_ref[1]
    ea = ea_ref[0] + ea_ref[1]
    h = jnp.maximum(
        jnp.dot(h_ref[...], wroot_ref[...], preferred_element_type=jnp.float32)
        + agg
        + jnp.dot(ea, wedge_ref[...], preferred_element_type=jnp.float32)
        + b_ref[...], 0.0)
    h_out[...] = h
    hm_out[...] = jnp.dot(h, wmsg_ref[...], preferred_element_type=jnp.float32)


ROWT = 2000
NT = N // ROWT


def _pool_body(h2_ref, p2_ref, pe_ref, batch_ref, fp_ref,
               wroot_ref, wedge_ref, b2_ref,
               wl0_ref, bl0_ref, wl1_ref, bl1_ref, wo_ref, bo_ref,
               h3_ref, y_ref, embed_ref, acc_ref):
    i = pl.program_id(0)
    agg = p2_ref[0, :, 0:1] + p2_ref[1, :, 0:1]
    ea = pe_ref[0] + pe_ref[1]
    h3 = jnp.maximum(
        jnp.dot(h2_ref[...], wroot_ref[...], preferred_element_type=jnp.float32)
        + agg
        + jnp.dot(ea, wedge_ref[...], preferred_element_type=jnp.float32)
        + b2_ref[...], 0.0)
    h3_ref[...] = h3
    bt = batch_ref[...].reshape(1, ROWT)
    gid = lax.broadcasted_iota(jnp.int32, (G, ROWT), 0)
    mask = (gid == bt).astype(jnp.float32)
    hfp = h3 * fp_ref[...]
    contrib = jnp.dot(mask, hfp, preferred_element_type=jnp.float32)

    @pl.when(i == 0)
    def _():
        acc_ref[...] = contrib

    @pl.when(i > 0)
    def _():
        acc_ref[...] += contrib

    @pl.when(i == NT - 1)
    def _():
        e = acc_ref[...]
        embed_ref[...] = e
        y = jnp.maximum(
            jnp.dot(e, wl0_ref[...], preferred_element_type=jnp.float32)
            + bl0_ref[...], 0.0)
        y = jnp.maximum(
            jnp.dot(y, wl1_ref[...], preferred_element_type=jnp.float32)
            + bl1_ref[...], 0.0)
        y_ref[...] = (jnp.dot(y, wo_ref[...], preferred_element_type=jnp.float32)
                      + bo_ref[...])


def kernel(x, edge_index, edge_attr, batch, fp,
           Wroot0, Wmsg0, Wedge0, b0,
           Wroot1, Wmsg1, Wedge1, b1,
           Wroot2, Wmsg2, Wedge2, b2,
           W_lin0, b_lin0, W_lin1, b_lin1, W_out, b_out):
    src3 = edge_index[0].reshape(NW, NCHUNK, K).astype(jnp.int32)
    dst3 = edge_index[1].reshape(NW, NCHUNK, K).astype(jnp.int32)

    hm0 = pl.pallas_call(
        _mm0_body,
        out_shape=jax.ShapeDtypeStruct((N, 64), jnp.float32),
    )(x, Wmsg0)
    p0 = _seg_partials(hm0, src3, dst3, NCHUNK, K)
    eap = _seg_partials(None, src3, dst3, NCHUNK, K, edge_rows=edge_attr)
    h1, hm1 = pl.pallas_call(
        _combine_body,
        out_shape=(jax.ShapeDtypeStruct((N, 64), jnp.float32),
                   jax.ShapeDtypeStruct((N, 32), jnp.float32)),
    )(x, p0, eap, Wroot0, Wedge0, b0.reshape(1, 64), Wmsg1)

    p1 = _seg_partials(hm1, src3, dst3, NCHUNK, K)
    Wmsg2b = jnp.tile(Wmsg2, (1, 16))
    h2, hm2b = pl.pallas_call(
        _combine_body,
        out_shape=(jax.ShapeDtypeStruct((N, 32), jnp.float32),
                   jax.ShapeDtypeStruct((N, 16), jnp.float32)),
    )(h1, p1, eap, Wroot1, Wedge1, b1.reshape(1, 32), Wmsg2b)

    p2 = _seg_partials(hm2b, src3, dst3, NCHUNK, K)

    batch3 = batch.reshape(NT, 1, ROWT).astype(jnp.int32)
    h3, y, embed = pl.pallas_call(
        _pool_body,
        grid=(NT,),
        in_specs=[
            pl.BlockSpec((ROWT, 32), lambda i: (i, 0)),
            pl.BlockSpec((2, ROWT, 16), lambda i: (0, i, 0)),
            pl.BlockSpec((2, ROWT, 16), lambda i: (0, i, 0)),
            pl.BlockSpec((1, 1, ROWT), lambda i: (i, 0, 0)),
            pl.BlockSpec((ROWT, FP), lambda i: (i, 0)),
            pl.BlockSpec((32, 1), lambda i: (0, 0)),
            pl.BlockSpec((16, 1), lambda i: (0, 0)),
            pl.BlockSpec((1, 1), lambda i: (0, 0)),
            pl.BlockSpec((FP, 256), lambda i: (0, 0)),
            pl.BlockSpec((1, 256), lambda i: (0, 0)),
            pl.BlockSpec((256, 64), lambda i: (0, 0)),
            pl.BlockSpec((1, 64), lambda i: (0, 0)),
            pl.BlockSpec((64, 1), lambda i: (0, 0)),
            pl.BlockSpec((1, 1), lambda i: (0, 0)),
        ],
        out_specs=[
            pl.BlockSpec((ROWT, 1), lambda i: (i, 0)),
            pl.BlockSpec((G, 1), lambda i: (0, 0)),
            pl.BlockSpec((G, FP), lambda i: (0, 0)),
        ],
        out_shape=[
            jax.ShapeDtypeStruct((N, 1), jnp.float32),
            jax.ShapeDtypeStruct((G, 1), jnp.float32),
            jax.ShapeDtypeStruct((G, FP), jnp.float32),
        ],
        scratch_shapes=[pltpu.VMEM((G, FP), jnp.float32)],
    )(h2, p2, eap, batch3, fp,
      Wroot2, Wedge2, b2.reshape(1, 1),
      W_lin0, b_lin0.reshape(1, 256), W_lin1, b_lin1.reshape(1, 64),
      W_out, b_out.reshape(1, 1))

    return (h3, y, embed)


# speedup vs baseline: 1.0148x; 1.0148x over previous
import functools

import jax
import jax.numpy as jnp
from jax import lax
from jax.experimental import pallas as pl
from jax.experimental.pallas import tpu as pltpu
from jax.experimental.pallas import tpu_sc as plsc

N = 10000
E = 320000
D = 128
ED = 16
FP = 881
G = 512

NC = 2
NS = 16
NW = NC * NS
EPW = E // NW
K = 125
NCHUNK = EPW // K
NPS = 640
NPS_LAST = N - (NS - 1) * NPS

_SC_MESH = plsc.VectorSubcoreMesh(core_axis_name="c", subcore_axis_name="s")


NB = 8
LAG = 4


def _seg_partials(table, src3, dst3, nchunk, k, edge_rows=None):
    gather = table is not None
    W = table.shape[-1] if gather else edge_rows.shape[-1]
    assert nchunk * k == EPW and k <= 128
    zeros_nw = jnp.zeros((N, W), jnp.float32)
    if edge_rows is None:
        edge_rows = jnp.zeros((8, 16), jnp.float32)
    if table is None:
        table = jnp.zeros((8, 16), jnp.float32)
    @functools.partial(
        pl.kernel,
        mesh=_SC_MESH,
        out_type=jax.ShapeDtypeStruct((NC, N, W), jnp.float32),
        scratch_types=[
            pltpu.VMEM((nchunk, k), jnp.int32),
            pltpu.VMEM((nchunk, k), jnp.int32),
            *[pltpu.VMEM((k, W), jnp.float32) for _ in range(NB)],
            pltpu.VMEM_SHARED((N, W), jnp.float32),
            *[pltpu.SemaphoreType.DMA for _ in range(2 * NB)],
        ],
        compiler_params=pltpu.CompilerParams(use_tc_tiling_on_sc=False),
    )
    def sc_kernel(table_hbm, src_hbm, dst_hbm, zero_hbm, ea_hbm, out_hbm,
                  sidx, didx, *rest):
        rows = rest[:NB]
        acc = rest[NB]
        gsem = rest[NB + 1:NB + 1 + NB]
        ssem = rest[NB + 1 + NB:]
        c = lax.axis_index("c")
        s = lax.axis_index("s")
        wid = s * NC + c

        def for_slices(fn):
            @pl.when(s < NS - 1)
            def _():
                fn(pl.ds(s * NPS, NPS), NPS)

            @pl.when(s == NS - 1)
            def _():
                fn(pl.ds((NS - 1) * NPS, NPS_LAST), NPS_LAST)

        for_slices(lambda sl, n: pltpu.sync_copy(zero_hbm.at[sl],
                                                 acc.at[sl]))
        if gather:
            pltpu.sync_copy(src_hbm.at[wid], sidx)
        pltpu.sync_copy(dst_hbm.at[wid], didx)
        plsc.subcore_barrier()

        def fetch_pair(i, b):
            if gather:
                return table_hbm.at[sidx.at[i]], rows[b]
            return ea_hbm.at[pl.ds(wid * EPW + i * k, k)], rows[b]

        def fetch(i, b):
            s_, d_ = fetch_pair(i, b)
            pltpu.async_copy(s_, d_, gsem[b])

        def fetch_wait(i, b):
            s_, d_ = fetch_pair(i, b)
            pltpu.make_async_copy(s_, d_, gsem[b]).wait()

        def scat(i, b):
            pltpu.async_copy(rows[b], acc.at[didx.at[i]], ssem[b], add=True)

        def scat_wait(i, b):
            pltpu.make_async_copy(rows[b], acc.at[didx.at[i]], ssem[b]).wait()

        def process(cc):
            fetch_wait(cc, cc % NB)
            scat(cc, cc % NB)
            if cc + LAG < nchunk:
                scat_wait(cc - LAG, (cc - LAG) % NB)
                fetch(cc + LAG, (cc + LAG) % NB)

        main_iters = (nchunk - 2 * LAG) // NB
        main_end = LAG + main_iters * NB

        for cc in range(NB):
            fetch(cc, cc)
        for cc in range(LAG):
            fetch_wait(cc, cc)
            scat(cc, cc)

        @pl.loop(LAG, main_end, step=NB)
        def _(i0):
            for j in range(NB):
                cc = i0 + j
                b = (LAG + j) % NB
                fetch_wait(cc, b)
                scat(cc, b)
                scat_wait(cc - LAG, j)
                fetch(cc + LAG, j)

        for cc in range(main_end, nchunk):
            process(cc)
        for cc in range(nchunk - 2 * LAG, nchunk):
            scat_wait(cc, cc % NB)

        plsc.subcore_barrier()
        for_slices(lambda sl, n: pltpu.sync_copy(acc.at[sl],
                                                 out_hbm.at[c].at[sl]))

    return sc_kernel(table, src3, dst3, zeros_nw, edge_rows)


def _mm0_body(x_ref, w_ref, o_ref):
    o_ref[...] = jnp.dot(x_ref[...], w_ref[...],
                         preferred_element_type=jnp.float32)


def _combine_body(h_ref, p_ref, ea_ref, wroot_ref, wedge_ref, b_ref,
                  wmsg_ref, h_out, hm_out):
    agg = p_ref[0] + p_ref[1]
    ea = ea_ref[0] + ea_ref[1]
    h = jnp.maximum(
        jnp.dot(h_ref[...], wroot_ref[...], preferred_element_type=jnp.float32)
        + agg
        + jnp.dot(ea, wedge_ref[...], preferred_element_type=jnp.float32)
        + b_ref[...], 0.0)
    h_out[...] = h
    hm_out[...] = jnp.dot(h, wmsg_ref[...], preferred_element_type=jnp.float32)


ROWT = 2000
NT = N // ROWT


def _pool_body(h2_ref, p2_ref, pe_ref, batch_ref, fp_ref,
               wroot_ref, wedge_ref, b2_ref,
               wl0_ref, bl0_ref, wl1_ref, bl1_ref, wo_ref, bo_ref,
               h3_ref, y_ref, embed_ref, acc_ref):
    i = pl.program_id(0)
    agg = p2_ref[0, :, 0:1] + p2_ref[1, :, 0:1]
    ea = pe_ref[0] + pe_ref[1]
    h3 = jnp.maximum(
        jnp.dot(h2_ref[...], wroot_ref[...], preferred_element_type=jnp.float32)
        + agg
        + jnp.dot(ea, wedge_ref[...], preferred_element_type=jnp.float32)
        + b2_ref[...], 0.0)
    h3_ref[...] = h3
    bt = batch_ref[...].reshape(1, ROWT)
    gid = lax.broadcasted_iota(jnp.int32, (G, ROWT), 0)
    mask = (gid == bt).astype(jnp.float32)
    hfp = h3 * fp_ref[...]
    contrib = jnp.dot(mask, hfp, preferred_element_type=jnp.float32)

    @pl.when(i == 0)
    def _():
        acc_ref[...] = contrib

    @pl.when(i > 0)
    def _():
        acc_ref[...] += contrib

    @pl.when(i == NT - 1)
    def _():
        e = acc_ref[...]
        embed_ref[...] = e
        y = jnp.maximum(
            jnp.dot(e, wl0_ref[...], preferred_element_type=jnp.float32)
            + bl0_ref[...], 0.0)
        y = jnp.maximum(
            jnp.dot(y, wl1_ref[...], preferred_element_type=jnp.float32)
            + bl1_ref[...], 0.0)
        y_ref[...] = (jnp.dot(y, wo_ref[...], preferred_element_type=jnp.float32)
                      + bo_ref[...])


def kernel(x, edge_index, edge_attr, batch, fp,
           Wroot0, Wmsg0, Wedge0, b0,
           Wroot1, Wmsg1, Wedge1, b1,
           Wroot2, Wmsg2, Wedge2, b2,
           W_lin0, b_lin0, W_lin1, b_lin1, W_out, b_out):
    src3 = edge_index[0].reshape(NW, NCHUNK, K).astype(jnp.int32)
    dst3 = edge_index[1].reshape(NW, NCHUNK, K).astype(jnp.int32)

    hm0 = pl.pallas_call(
        _mm0_body,
        out_shape=jax.ShapeDtypeStruct((N, 64), jnp.float32),
    )(x, Wmsg0)
    p0 = _seg_partials(hm0, src3, dst3, NCHUNK, K)
    eap = _seg_partials(None, src3, dst3, NCHUNK, K, edge_rows=edge_attr)
    h1, hm1 = pl.pallas_call(
        _combine_body,
        out_shape=(jax.ShapeDtypeStruct((N, 64), jnp.float32),
                   jax.ShapeDtypeStruct((N, 32), jnp.float32)),
    )(x, p0, eap, Wroot0, Wedge0, b0.reshape(1, 64), Wmsg1)

    p1 = _seg_partials(hm1, src3, dst3, NCHUNK, K)
    Wmsg2b = jnp.tile(Wmsg2, (1, 16))
    h2, hm2b = pl.pallas_call(
        _combine_body,
        out_shape=(jax.ShapeDtypeStruct((N, 32), jnp.float32),
                   jax.ShapeDtypeStruct((N, 16), jnp.float32)),
    )(h1, p1, eap, Wroot1, Wedge1, b1.reshape(1, 32), Wmsg2b)

    p2 = _seg_partials(hm2b, src3, dst3, NCHUNK, K)

    batch3 = batch.reshape(NT, 1, ROWT).astype(jnp.int32)
    h3, y, embed = pl.pallas_call(
        _pool_body,
        grid=(NT,),
        in_specs=[
            pl.BlockSpec((ROWT, 32), lambda i: (i, 0)),
            pl.BlockSpec((2, ROWT, 16), lambda i: (0, i, 0)),
            pl.BlockSpec((2, ROWT, 16), lambda i: (0, i, 0)),
            pl.BlockSpec((1, 1, ROWT), lambda i: (i, 0, 0)),
            pl.BlockSpec((ROWT, FP), lambda i: (i, 0)),
            pl.BlockSpec((32, 1), lambda i: (0, 0)),
            pl.BlockSpec((16, 1), lambda i: (0, 0)),
            pl.BlockSpec((1, 1), lambda i: (0, 0)),
            pl.BlockSpec((FP, 256), lambda i: (0, 0)),
            pl.BlockSpec((1, 256), lambda i: (0, 0)),
            pl.BlockSpec((256, 64), lambda i: (0, 0)),
            pl.BlockSpec((1, 64), lambda i: (0, 0)),
            pl.BlockSpec((64, 1), lambda i: (0, 0)),
            pl.BlockSpec((1, 1), lambda i: (0, 0)),
        ],
        out_specs=[
            pl.BlockSpec((ROWT, 1), lambda i: (i, 0)),
            pl.BlockSpec((G, 1), lambda i: (0, 0)),
            pl.BlockSpec((G, FP), lambda i: (0, 0)),
        ],
        out_shape=[
            jax.ShapeDtypeStruct((N, 1), jnp.float32),
            jax.ShapeDtypeStruct((G, 1), jnp.float32),
            jax.ShapeDtypeStruct((G, FP), jnp.float32),
        ],
        scratch_shapes=[pltpu.VMEM((G, FP), jnp.float32)],
    )(h2, p2, eap, batch3, fp,
      Wroot2, Wedge2, b2.reshape(1, 1),
      W_lin0, b_lin0.reshape(1, 256), W_lin1, b_lin1.reshape(1, 64),
      W_out, b_out.reshape(1, 1))

    return (h3, y, embed)
